# Initial kernel scaffold; baseline (speedup 1.0000x reference)
import jax
import jax.numpy as jnp
from jax.experimental import pallas as pl


def kernel(x, edge_index, W_self1, W_neigh1, b1, gamma1, beta1, W_self2, W_neigh2, b2, gamma2, beta2, W_self3, W_neigh3, b3):
    raise NotImplementedError("write your pallas kernel here")


# speedup vs baseline: 4.3505x; 4.3505x over previous
import functools

import jax
import jax.numpy as jnp
from jax import lax
from jax.experimental import pallas as pl
from jax.experimental.pallas import tpu as pltpu
from jax.experimental.pallas import tpu_sc as plsc

N = 10000
E = 320000
F = 128
EPS = 1e-5

NC = 2
NS = 16
NW = NC * NS
EDGES_PER_W = E // NW
CHUNK = 80
NCHUNK = EDGES_PER_W // CHUNK
N_PAD = 10240
ROWS_PER_TILE = N_PAD // NS


def _mesh():
    return plsc.VectorSubcoreMesh(core_axis_name="c", subcore_axis_name="s")


@functools.partial(
    pl.kernel,
    out_type=jax.ShapeDtypeStruct((NC * N_PAD, F), jnp.float32),
    mesh=_mesh(),
    scratch_types=[
        pltpu.VMEM((CHUNK,), jnp.int32),
        pltpu.VMEM((CHUNK, F), jnp.float32),
        pltpu.VMEM_SHARED((N_PAD, F), jnp.float32),
    ],
)
def _sc_degree(dst_hbm, zeros_hbm, ones_hbm, out_hbm, dst_v, ones_v,
               shared_deg):
    c = lax.axis_index("c")
    s = lax.axis_index("s")
    wid = c * NS + s

    pltpu.sync_copy(
        zeros_hbm.at[pl.ds(s * ROWS_PER_TILE, ROWS_PER_TILE)],
        shared_deg.at[pl.ds(s * ROWS_PER_TILE, ROWS_PER_TILE)],
    )
    pltpu.sync_copy(ones_hbm, ones_v)
    plsc.subcore_barrier()

    def body(i, carry):
        base = wid * EDGES_PER_W + i * CHUNK
        pltpu.sync_copy(dst_hbm.at[pl.ds(base, CHUNK)], dst_v)
        pltpu.sync_copy(ones_v, shared_deg.at[dst_v], add=True)
        return carry

    lax.fori_loop(0, NCHUNK, body, 0)
    plsc.subcore_barrier()

    pltpu.sync_copy(
        shared_deg.at[pl.ds(s * ROWS_PER_TILE, ROWS_PER_TILE)],
        out_hbm.at[pl.ds(c * N_PAD + s * ROWS_PER_TILE, ROWS_PER_TILE)],
    )


@functools.partial(
    pl.kernel,
    out_type=jax.ShapeDtypeStruct((NC * N_PAD, F), jnp.float32),
    mesh=_mesh(),
    scratch_types=[
        pltpu.VMEM((CHUNK,), jnp.int32),
        pltpu.VMEM((CHUNK,), jnp.int32),
        pltpu.VMEM((CHUNK, F), jnp.float32),
        pltpu.VMEM_SHARED((N_PAD, F), jnp.float32),
        pltpu.SemaphoreType.DMA,
    ],
)
def _sc_agg(h_hbm, src_hbm, dst_hbm, zeros_hbm, out_hbm,
            src_v, dst_v, rows_v, shared_agg, sem):
    c = lax.axis_index("c")
    s = lax.axis_index("s")
    wid = c * NS + s

    pltpu.sync_copy(
        zeros_hbm.at[pl.ds(s * ROWS_PER_TILE, ROWS_PER_TILE)],
        shared_agg.at[pl.ds(s * ROWS_PER_TILE, ROWS_PER_TILE)],
    )
    plsc.subcore_barrier()

    def body(i, carry):
        base = wid * EDGES_PER_W + i * CHUNK
        pltpu.sync_copy(src_hbm.at[pl.ds(base, CHUNK)], src_v)
        pltpu.sync_copy(dst_hbm.at[pl.ds(base, CHUNK)], dst_v)
        pltpu.async_copy(h_hbm.at[src_v], rows_v, sem).wait()
        pltpu.sync_copy(rows_v, shared_agg.at[dst_v], add=True)
        return carry

    lax.fori_loop(0, NCHUNK, body, 0)
    plsc.subcore_barrier()

    pltpu.sync_copy(
        shared_agg.at[pl.ds(s * ROWS_PER_TILE, ROWS_PER_TILE)],
        out_hbm.at[pl.ds(c * N_PAD + s * ROWS_PER_TILE, ROWS_PER_TILE)],
    )


def _tc_dense_bn(h, agg2, deg2, w_self, w_neigh, b, gamma, beta):
    def body(h_ref, agg_ref, deg_ref, ws_ref, wn_ref, b_ref, g_ref, be_ref,
             o_ref):
        deg = deg_ref[0:N, :] + deg_ref[N_PAD:N_PAD + N, :]
        agg = agg_ref[0:N, :] + agg_ref[N_PAD:N_PAD + N, :]
        hn = agg / jnp.maximum(deg, 1.0)
        z = (jnp.dot(h_ref[...], ws_ref[...], preferred_element_type=jnp.float32)
             + jnp.dot(hn, wn_ref[...], preferred_element_type=jnp.float32)
             + b_ref[...])
        mu = jnp.mean(z, axis=0, keepdims=True)
        var = jnp.mean((z - mu) ** 2, axis=0, keepdims=True)
        z = (z - mu) * lax.rsqrt(var + EPS) * g_ref[...] + be_ref[...]
        o_ref[...] = jnp.maximum(z, 0.0)

    return pl.pallas_call(
        body,
        out_shape=jax.ShapeDtypeStruct((N, F), jnp.float32),
    )(h, agg2, deg2, w_self, w_neigh, b.reshape(1, F), gamma.reshape(1, F),
      beta.reshape(1, F))


def _tc_dense(h, agg2, deg2, w_self, w_neigh, b):
    def body(h_ref, agg_ref, deg_ref, ws_ref, wn_ref, b_ref, o_ref):
        deg = deg_ref[0:N, :] + deg_ref[N_PAD:N_PAD + N, :]
        agg = agg_ref[0:N, :] + agg_ref[N_PAD:N_PAD + N, :]
        hn = agg / jnp.maximum(deg, 1.0)
        o_ref[...] = (
            jnp.dot(h_ref[...], ws_ref[...], preferred_element_type=jnp.float32)
            + jnp.dot(hn, wn_ref[...], preferred_element_type=jnp.float32)
            + b_ref[...])

    return pl.pallas_call(
        body,
        out_shape=jax.ShapeDtypeStruct((N, F), jnp.float32),
    )(h, agg2, deg2, w_self, w_neigh, b.reshape(1, F))


def kernel(x, edge_index, W_self1, W_neigh1, b1, gamma1, beta1,
           W_self2, W_neigh2, b2, gamma2, beta2,
           W_self3, W_neigh3, b3):
    src = edge_index[0]
    dst = edge_index[1]
    zeros_nf = jnp.zeros((N_PAD, F), jnp.float32)
    ones_cf = jnp.ones((CHUNK, F), jnp.float32)

    deg2 = _sc_degree(dst, zeros_nf, ones_cf)[:, 0:1]
    agg2 = _sc_agg(x, src, dst, zeros_nf)
    h = _tc_dense_bn(x, agg2, deg2, W_self1, W_neigh1, b1, gamma1, beta1)
    agg2 = _sc_agg(h, src, dst, zeros_nf)
    h = _tc_dense_bn(h, agg2, deg2, W_self2, W_neigh2, b2, gamma2, beta2)
    agg2 = _sc_agg(h, src, dst, zeros_nf)
    h = _tc_dense(h, agg2, deg2, W_self3, W_neigh3, b3)
    return h


# speedup vs baseline: 7.9128x; 1.8188x over previous
import functools

import jax
import jax.numpy as jnp
from jax import lax
from jax.experimental import pallas as pl
from jax.experimental.pallas import tpu as pltpu
from jax.experimental.pallas import tpu_sc as plsc

N = 10000
E = 320000
F = 128
EPS = 1e-5

NC = 2
NS = 16
NW = NC * NS
EDGES_PER_W = E // NW
CHUNK = 80
NCHUNK = EDGES_PER_W // CHUNK
N_PAD = 10240
ROWS_PER_TILE = N_PAD // NS


def _mesh():
    return plsc.VectorSubcoreMesh(core_axis_name="c", subcore_axis_name="s")


@functools.partial(
    pl.kernel,
    out_type=jax.ShapeDtypeStruct((NC * N_PAD, F), jnp.float32),
    mesh=_mesh(),
    scratch_types=[
        pltpu.VMEM((NCHUNK, CHUNK), jnp.int32),
        pltpu.VMEM((CHUNK, F), jnp.float32),
        pltpu.VMEM_SHARED((N_PAD, F), jnp.float32),
    ],
)
def _sc_degree(dst_hbm, zeros_hbm, ones_hbm, out_hbm, dst_v, ones_v,
               shared_deg):
    c = lax.axis_index("c")
    s = lax.axis_index("s")
    wid = c * NS + s

    pltpu.sync_copy(
        zeros_hbm.at[pl.ds(s * ROWS_PER_TILE, ROWS_PER_TILE)],
        shared_deg.at[pl.ds(s * ROWS_PER_TILE, ROWS_PER_TILE)],
    )
    pltpu.sync_copy(ones_hbm, ones_v)
    pltpu.sync_copy(dst_hbm.at[wid], dst_v)
    plsc.subcore_barrier()

    def body(i, carry):
        pltpu.sync_copy(ones_v, shared_deg.at[dst_v.at[i]], add=True)
        return carry

    lax.fori_loop(0, NCHUNK, body, 0)
    plsc.subcore_barrier()

    pltpu.sync_copy(
        shared_deg.at[pl.ds(s * ROWS_PER_TILE, ROWS_PER_TILE)],
        out_hbm.at[pl.ds(c * N_PAD + s * ROWS_PER_TILE, ROWS_PER_TILE)],
    )


@functools.partial(
    pl.kernel,
    out_type=jax.ShapeDtypeStruct((NC * N_PAD, F), jnp.float32),
    mesh=_mesh(),
    scratch_types=[
        pltpu.VMEM(((NCHUNK + 1) * CHUNK,), jnp.int32),
        pltpu.VMEM((NCHUNK, CHUNK), jnp.int32),
        pltpu.VMEM((CHUNK, F), jnp.float32),
        pltpu.VMEM((CHUNK, F), jnp.float32),
        pltpu.VMEM_SHARED((N_PAD, F), jnp.float32),
        pltpu.SemaphoreType.DMA,
        pltpu.SemaphoreType.DMA,
    ],
)
def _sc_agg(h_hbm, src_hbm, dst_hbm, zeros_hbm, out_hbm,
            src_v, dst_v, rows_a, rows_b, shared_agg, sem_a, sem_b):
    c = lax.axis_index("c")
    s = lax.axis_index("s")
    wid = c * NS + s

    pltpu.sync_copy(
        zeros_hbm.at[pl.ds(s * ROWS_PER_TILE, ROWS_PER_TILE)],
        shared_agg.at[pl.ds(s * ROWS_PER_TILE, ROWS_PER_TILE)],
    )
    pltpu.sync_copy(src_hbm.at[pl.ds(wid * (NCHUNK + 1) * CHUNK,
                                     (NCHUNK + 1) * CHUNK)], src_v)
    pltpu.sync_copy(dst_hbm.at[wid], dst_v)
    plsc.subcore_barrier()

    pltpu.async_copy(h_hbm.at[src_v.at[pl.ds(0, CHUNK)]], rows_a, sem_a)

    def step(i, rows_cur, sem_cur, rows_nxt, sem_nxt):
        pltpu.make_async_copy(h_hbm.at[src_v.at[pl.ds(i * CHUNK, CHUNK)]], rows_cur, sem_cur).wait()
        pltpu.async_copy(h_hbm.at[src_v.at[pl.ds((i + 1) * CHUNK, CHUNK)]], rows_nxt, sem_nxt)
        pltpu.sync_copy(rows_cur, shared_agg.at[dst_v.at[i]], add=True)

    def body(i, carry):
        @pl.when(i % 2 == 0)
        def _():
            step(i, rows_a, sem_a, rows_b, sem_b)

        @pl.when(i % 2 == 1)
        def _():
            step(i, rows_b, sem_b, rows_a, sem_a)

        return carry

    lax.fori_loop(0, NCHUNK, body, 0)
    last = rows_a if NCHUNK % 2 == 0 else rows_b
    last_sem = sem_a if NCHUNK % 2 == 0 else sem_b
    pltpu.make_async_copy(h_hbm.at[src_v.at[pl.ds(NCHUNK * CHUNK, CHUNK)]], last, last_sem).wait()
    plsc.subcore_barrier()

    pltpu.sync_copy(
        shared_agg.at[pl.ds(s * ROWS_PER_TILE, ROWS_PER_TILE)],
        out_hbm.at[pl.ds(c * N_PAD + s * ROWS_PER_TILE, ROWS_PER_TILE)],
    )


def _tc_dense_bn(h, agg2, deg2, w_self, w_neigh, b, gamma, beta):
    def body(h_ref, agg_ref, deg_ref, ws_ref, wn_ref, b_ref, g_ref, be_ref,
             o_ref):
        deg = deg_ref[0:N, :] + deg_ref[N_PAD:N_PAD + N, :]
        agg = agg_ref[0:N, :] + agg_ref[N_PAD:N_PAD + N, :]
        hn = agg / jnp.maximum(deg, 1.0)
        z = (jnp.dot(h_ref[...], ws_ref[...], preferred_element_type=jnp.float32)
             + jnp.dot(hn, wn_ref[...], preferred_element_type=jnp.float32)
             + b_ref[...])
        mu = jnp.mean(z, axis=0, keepdims=True)
        var = jnp.mean((z - mu) ** 2, axis=0, keepdims=True)
        z = (z - mu) * lax.rsqrt(var + EPS) * g_ref[...] + be_ref[...]
        o_ref[...] = jnp.maximum(z, 0.0)

    return pl.pallas_call(
        body,
        out_shape=jax.ShapeDtypeStruct((N, F), jnp.float32),
    )(h, agg2, deg2, w_self, w_neigh, b.reshape(1, F), gamma.reshape(1, F),
      beta.reshape(1, F))


def _tc_dense(h, agg2, deg2, w_self, w_neigh, b):
    def body(h_ref, agg_ref, deg_ref, ws_ref, wn_ref, b_ref, o_ref):
        deg = deg_ref[0:N, :] + deg_ref[N_PAD:N_PAD + N, :]
        agg = agg_ref[0:N, :] + agg_ref[N_PAD:N_PAD + N, :]
        hn = agg / jnp.maximum(deg, 1.0)
        o_ref[...] = (
            jnp.dot(h_ref[...], ws_ref[...], preferred_element_type=jnp.float32)
            + jnp.dot(hn, wn_ref[...], preferred_element_type=jnp.float32)
            + b_ref[...])

    return pl.pallas_call(
        body,
        out_shape=jax.ShapeDtypeStruct((N, F), jnp.float32),
    )(h, agg2, deg2, w_self, w_neigh, b.reshape(1, F))


def kernel(x, edge_index, W_self1, W_neigh1, b1, gamma1, beta1,
           W_self2, W_neigh2, b2, gamma2, beta2,
           W_self3, W_neigh3, b3):
    src3 = edge_index[0].reshape(NW, NCHUNK, CHUNK)
    src3 = jnp.concatenate([src3, src3[:, :1, :]], axis=1).reshape(-1)
    dst3 = edge_index[1].reshape(NW, NCHUNK, CHUNK)
    zeros_nf = jnp.zeros((N_PAD, F), jnp.float32)
    ones_cf = jnp.ones((CHUNK, F), jnp.float32)

    deg2 = _sc_degree(dst3, zeros_nf, ones_cf)[:, 0:1]
    agg2 = _sc_agg(x, src3, dst3, zeros_nf)
    h = _tc_dense_bn(x, agg2, deg2, W_self1, W_neigh1, b1, gamma1, beta1)
    agg2 = _sc_agg(h, src3, dst3, zeros_nf)
    h = _tc_dense_bn(h, agg2, deg2, W_self2, W_neigh2, b2, gamma2, beta2)
    agg2 = _sc_agg(h, src3, dst3, zeros_nf)
    h = _tc_dense(h, agg2, deg2, W_self3, W_neigh3, b3)
    return h
